# baseline (speedup 1.0000x reference)
import functools

import jax
import jax.numpy as jnp
from jax import lax
from jax.experimental import pallas as pl
from jax.experimental.pallas import tpu as pltpu
from jax.experimental.pallas import tpu_sc as plsc

PW = 4
NC, NS = 2, 16
NW = NC * NS
L = 16


def _tc_body(P, rsel_ref, csel_ref, x_ref, nhwc_ref, invt_ref):
    x3 = x_ref[0]
    H, W = x3.shape[1], x3.shape[2]
    nhwc_ref[...] = jnp.transpose(x3, (1, 2, 0)).reshape(H * W, x3.shape[0])
    cs = jnp.sum(x3 * x3, axis=0)
    tmp = cs
    for j in range(1, PW):
        tmp = tmp + jnp.roll(cs, -j, axis=1)
    win = tmp
    for i in range(1, PW):
        win = win + jnp.roll(tmp, -i, axis=0)
    rows = lax.dot_general(rsel_ref[...], win, (((1,), (0,)), ((), ())),
                           preferred_element_type=jnp.float32)
    ss = jnp.sum(rows * csel_ref[...], axis=1, keepdims=True)
    inv = 1.0 / (jnp.sqrt(ss) + 1e-7)
    invt_ref[...] = jnp.broadcast_to(inv, (P, L))


def _sc_body(cdim, ppc, chunks_per_worker,
             nhwc, idxt, invt, out, idx_v, inv_v, rows_v, out_v, sem):
    cid = lax.axis_index("c")
    sid = lax.axis_index("s")
    wid = sid * NC + cid
    ncc = cdim // L
    n_rows = ppc * PW * PW

    def chunk(c, carry):
        m = wid * chunks_per_worker + c
        pltpu.sync_copy(idxt.at[m], idx_v)
        pltpu.sync_copy(invt.at[pl.ds(m * ppc, ppc)], inv_v)
        pltpu.async_copy(nhwc.at[idx_v], rows_v, sem).wait()
        for pi in range(ppc):
            inv = inv_v[pi, pl.ds(0, L)]

            def scale_row(i, carry2):
                rw = pi * 16 + i
                for cc in range(ncc):
                    v = rows_v[rw, pl.ds(cc * L, L)]
                    out_v[pi, pl.ds(i * cdim + cc * L, L)] = v * inv
                return carry2
            lax.fori_loop(0, 16, scale_row, 0)
        pltpu.sync_copy(out_v, out.at[pl.ds(m * ppc, ppc)])
        return carry

    lax.fori_loop(0, chunks_per_worker, chunk, 0)


def kernel(feats, num_patches, patch_ids):
    B, C, H, W = feats.shape
    P = patch_ids.shape[0]
    hw = H * W
    D = PW * PW * C

    r = patch_ids[:, 0].astype(jnp.int32)
    c = patch_ids[:, 1].astype(jnp.int32)
    pos = (r * W + c).reshape(P, 1)
    k = jnp.arange(PW * PW, dtype=jnp.int32)
    offs = (k // PW) * W + (k % PW)
    idx = (jnp.arange(B, dtype=jnp.int32) * hw)[:, None, None] \
        + pos[None, :, :] + offs[None, None, :]

    total_rows = B * P * PW * PW
    n_chunk_rows = 128
    n_chunks = total_rows // n_chunk_rows
    chunks_per_worker = n_chunks // NW
    ppc = n_chunk_rows // (PW * PW)
    idxt = idx.reshape(n_chunks, n_chunk_rows)

    rsel = (r[:, None] == jnp.arange(H, dtype=jnp.int32)[None, :]
            ).astype(jnp.float32)
    csel = (c[:, None] == jnp.arange(W, dtype=jnp.int32)[None, :]
            ).astype(jnp.float32)

    nhwc, invt = pl.pallas_call(
        functools.partial(_tc_body, P),
        grid=(B,),
        in_specs=[
            pl.BlockSpec((P, H), lambda b: (0, 0)),
            pl.BlockSpec((P, W), lambda b: (0, 0)),
            pl.BlockSpec((1, C, H, W), lambda b: (b, 0, 0, 0)),
        ],
        out_specs=[
            pl.BlockSpec((hw, C), lambda b: (b, 0)),
            pl.BlockSpec((P, L), lambda b: (b, 0)),
        ],
        out_shape=[
            jax.ShapeDtypeStruct((B * hw, C), jnp.float32),
            jax.ShapeDtypeStruct((B * P, L), jnp.float32),
        ],
    )(rsel, csel, feats)

    mesh = plsc.VectorSubcoreMesh(core_axis_name="c", subcore_axis_name="s")
    out = pl.kernel(
        functools.partial(_sc_body, C, ppc, chunks_per_worker),
        out_type=jax.ShapeDtypeStruct((B * P, D), jnp.float32),
        mesh=mesh,
        scratch_types=[
            pltpu.VMEM((n_chunk_rows,), jnp.int32),
            pltpu.VMEM((ppc, L), jnp.float32),
            pltpu.VMEM((n_chunk_rows, C), jnp.float32),
            pltpu.VMEM((ppc, PW * PW * C), jnp.float32),
            pltpu.SemaphoreType.DMA,
        ],
    )(nhwc, idxt, invt)

    return (out, patch_ids)


# speedup vs baseline: 2.2541x; 2.2541x over previous
import functools

import jax
import jax.numpy as jnp
from jax import lax
from jax.experimental import pallas as pl
from jax.experimental.pallas import tpu as pltpu
from jax.experimental.pallas import tpu_sc as plsc

PW = 4
NC, NS = 2, 16
NW = NC * NS
L = 16


def _tc_body(W, P, pos_ref, x_ref, invt_ref):
    x = x_ref[...]
    hw = x.shape[0]
    rowsq = jnp.sum(x * x, axis=1, keepdims=True)
    tmp = rowsq
    for j in range(1, PW):
        tmp = tmp + jnp.roll(rowsq, -j, axis=0)
    win = tmp
    for i in range(1, PW):
        win = win + jnp.roll(tmp, -i * W, axis=0)
    lane = lax.broadcasted_iota(jnp.int32, (P, hw), 1)
    oh = jnp.where(lane == pos_ref[...], 1.0, 0.0)
    ss = lax.dot_general(oh, win, (((1,), (0,)), ((), ())),
                         preferred_element_type=jnp.float32)
    inv = 1.0 / (jnp.sqrt(ss) + 1e-7)
    invt_ref[...] = jnp.broadcast_to(inv, (P, L))


def _sc_body(cdim, n_chunk_rows, chunks_per_worker,
             table, idxt, invt, out, idx_v, inv_v, rows_v, sem):
    cid = lax.axis_index("c")
    sid = lax.axis_index("s")
    wid = sid * NC + cid
    ppc = n_chunk_rows // (PW * PW)
    ncc = cdim // L

    def chunk(c, carry):
        m = wid * chunks_per_worker + c
        pltpu.sync_copy(idxt.at[m], idx_v)
        pltpu.sync_copy(invt.at[pl.ds(m * ppc, ppc)], inv_v)
        pltpu.async_copy(table.at[idx_v], rows_v, sem).wait()
        for pi in range(ppc):
            inv = inv_v[pi, pl.ds(0, L)]

            def scale_row(i, carry2):
                rw = pi * 16 + i
                for cc in range(ncc):
                    sl = pl.ds(cc * L, L)
                    rows_v[rw, sl] = rows_v[rw, sl] * inv
                return carry2
            lax.fori_loop(0, 16, scale_row, 0)
        pltpu.sync_copy(rows_v, out.at[pl.ds(m * n_chunk_rows, n_chunk_rows)])
        return carry

    lax.fori_loop(0, chunks_per_worker, chunk, 0)


def kernel(feats, num_patches, patch_ids):
    B, C, H, W = feats.shape
    P = patch_ids.shape[0]
    hw = H * W
    D = PW * PW * C

    table = jnp.transpose(feats, (0, 2, 3, 1)).reshape(B * hw, C)

    r = patch_ids[:, 0].astype(jnp.int32)
    c = patch_ids[:, 1].astype(jnp.int32)
    pos = (r * W + c).reshape(P, 1)
    k = jnp.arange(PW * PW, dtype=jnp.int32)
    offs = (k // PW) * W + (k % PW)
    idx = (jnp.arange(B, dtype=jnp.int32) * hw)[:, None, None] \
        + pos[None, :, :] + offs[None, None, :]

    total_rows = B * P * PW * PW
    n_chunk_rows = 128
    n_chunks = total_rows // n_chunk_rows
    chunks_per_worker = n_chunks // NW
    ppc = n_chunk_rows // (PW * PW)
    idxt = idx.reshape(n_chunks, n_chunk_rows)

    invt = pl.pallas_call(
        functools.partial(_tc_body, W, P),
        grid=(B,),
        in_specs=[
            pl.BlockSpec((P, 1), lambda b: (0, 0)),
            pl.BlockSpec((hw, C), lambda b: (b, 0)),
        ],
        out_specs=pl.BlockSpec((P, L), lambda b: (b, 0)),
        out_shape=jax.ShapeDtypeStruct((B * P, L), jnp.float32),
    )(pos, table)

    mesh = plsc.VectorSubcoreMesh(core_axis_name="c", subcore_axis_name="s")
    out_tbl = pl.kernel(
        functools.partial(_sc_body, C, n_chunk_rows, chunks_per_worker),
        out_type=jax.ShapeDtypeStruct((total_rows, C), jnp.float32),
        mesh=mesh,
        scratch_types=[
            pltpu.VMEM((n_chunk_rows,), jnp.int32),
            pltpu.VMEM((ppc, L), jnp.float32),
            pltpu.VMEM((n_chunk_rows, C), jnp.float32),
            pltpu.SemaphoreType.DMA,
        ],
    )(table, idxt, invt)

    out = out_tbl.reshape(B * P, D)
    return (out, patch_ids)
